# baseline (speedup 1.0000x reference)
import functools

import jax
import jax.numpy as jnp
from jax import lax
from jax.experimental import pallas as pl
from jax.experimental.pallas import tpu as pltpu
from jax.experimental.pallas import tpu_sc as plsc

DIM = 64
N_HEADS = 26
BATCH = 16384
N_ROWS = BATCH * N_HEADS
NC, NS, L = 2, 16, 16
NW = NC * NS
ROWS_PER_W = N_ROWS // NW
CHUNK = 128
N_CHUNKS = ROWS_PER_W // CHUNK
VREGS_PER_CHUNK = CHUNK // L


def _sc_gather(ids_hbm, offs_hbm, table_hbm, out_hbm,
               idx_v, offs_v, rows_v, gsem):
    wid = lax.axis_index("s") * NC + lax.axis_index("c")
    pltpu.sync_copy(ids_hbm.at[wid], idx_v)
    pltpu.sync_copy(offs_hbm, offs_v)

    out_base = wid * ROWS_PER_W

    def chunk_body(j, carry):
        for k in range(VREGS_PER_CHUNK):
            sl = pl.ds(k * L, L)
            idx_v[j, sl] = idx_v[j, sl] + offs_v[j, sl]
        pltpu.async_copy(table_hbm.at[idx_v.at[j]], rows_v, gsem).wait()
        pltpu.sync_copy(rows_v, out_hbm.at[pl.ds(out_base + j * CHUNK, CHUNK)])
        return carry

    lax.fori_loop(0, N_CHUNKS, chunk_body, 0)


@functools.partial(jax.jit, static_argnames=())
def _run(ids, offs_tiled, table):
    mesh = plsc.VectorSubcoreMesh(core_axis_name="c", subcore_axis_name="s")
    f = pl.kernel(
        _sc_gather,
        out_type=jax.ShapeDtypeStruct((N_ROWS, DIM), jnp.float32),
        mesh=mesh,
        scratch_types=[
            pltpu.VMEM((N_CHUNKS, CHUNK), jnp.int32),
            pltpu.VMEM((N_CHUNKS, CHUNK), jnp.int32),
            pltpu.VMEM((CHUNK, DIM), jnp.float32),
            pltpu.SemaphoreType.DMA,
        ],
        compiler_params=pltpu.CompilerParams(use_tc_tiling_on_sc=False),
    )
    return f(ids, offs_tiled, table)


def kernel(input_ids, offsets, table):
    ids = input_ids.astype(jnp.int32).reshape(NW, N_CHUNKS, CHUNK)
    offs_tiled = jnp.tile(offsets.astype(jnp.int32),
                          ROWS_PER_W // N_HEADS).reshape(N_CHUNKS, CHUNK)
    out = _run(ids, offs_tiled, table.astype(jnp.float32))
    return out.reshape(BATCH, N_HEADS, DIM)


# speedup vs baseline: 1.0404x; 1.0404x over previous
import functools

import jax
import jax.numpy as jnp
from jax import lax
from jax.experimental import pallas as pl
from jax.experimental.pallas import tpu as pltpu
from jax.experimental.pallas import tpu_sc as plsc

DIM = 64
N_HEADS = 26
BATCH = 16384
N_ROWS = BATCH * N_HEADS
NC, NS, L = 2, 16, 16
NW = NC * NS
ROWS_PER_W = N_ROWS // NW
CHUNK = 128
N_CHUNKS = ROWS_PER_W // CHUNK
VREGS_PER_CHUNK = CHUNK // L
NBUF = 8
N_GROUPS = N_CHUNKS // NBUF


def _sc_gather(ids_hbm, offs_hbm, table_hbm, out_hbm,
               idx_v, offs_v, rows_v, gsem, osem):
    wid = lax.axis_index("s") * NC + lax.axis_index("c")
    pltpu.sync_copy(ids_hbm.at[wid], idx_v)
    pltpu.sync_copy(offs_hbm, offs_v)
    out_base = wid * ROWS_PER_W

    def add_offsets(j):
        for k in range(VREGS_PER_CHUNK):
            sl = pl.ds(k * L, L)
            idx_v[j, sl] = idx_v[j, sl] + offs_v[j, sl]

    def gather(j, b):
        return pltpu.make_async_copy(
            table_hbm.at[idx_v.at[j]], rows_v.at[b], gsem.at[b])

    def writeback(j, b):
        return pltpu.make_async_copy(
            rows_v.at[b], out_hbm.at[pl.ds(out_base + j * CHUNK, CHUNK)],
            osem.at[b])

    for b in range(NBUF):
        add_offsets(b)
        gather(b, b).start()

    def group_body(g, carry):
        for b in range(NBUF):
            j = g * NBUF + b
            gather(j, b).wait()
            writeback(j, b).start()
            jn = j + NBUF
            add_offsets(jn)
            writeback(j, b).wait()
            gather(jn, b).start()
        return carry

    lax.fori_loop(0, N_GROUPS - 1, group_body, 0)

    for b in range(NBUF):
        j = (N_GROUPS - 1) * NBUF + b
        gather(j, b).wait()
        writeback(j, b).start()
    for b in range(NBUF):
        j = (N_GROUPS - 1) * NBUF + b
        writeback(j, b).wait()


@jax.jit
def _run(ids, offs_tiled, table):
    mesh = plsc.VectorSubcoreMesh(core_axis_name="c", subcore_axis_name="s")
    f = pl.kernel(
        _sc_gather,
        out_type=jax.ShapeDtypeStruct((N_ROWS, DIM), jnp.float32),
        mesh=mesh,
        scratch_types=[
            pltpu.VMEM((N_CHUNKS, CHUNK), jnp.int32),
            pltpu.VMEM((N_CHUNKS, CHUNK), jnp.int32),
            pltpu.VMEM((NBUF, CHUNK, DIM), jnp.float32),
            pltpu.SemaphoreType.DMA((NBUF,)),
            pltpu.SemaphoreType.DMA((NBUF,)),
        ],
        compiler_params=pltpu.CompilerParams(use_tc_tiling_on_sc=False),
    )
    return f(ids, offs_tiled, table)


def kernel(input_ids, offsets, table):
    ids = input_ids.astype(jnp.int32).reshape(NW, N_CHUNKS, CHUNK)
    offs_tiled = jnp.tile(offsets.astype(jnp.int32),
                          ROWS_PER_W // N_HEADS).reshape(N_CHUNKS, CHUNK)
    out = _run(ids, offs_tiled, table.astype(jnp.float32))
    return out.reshape(BATCH, N_HEADS, DIM)


# speedup vs baseline: 1.1252x; 1.0815x over previous
import functools

import jax
import jax.numpy as jnp
from jax import lax
from jax.experimental import pallas as pl
from jax.experimental.pallas import tpu as pltpu
from jax.experimental.pallas import tpu_sc as plsc

DIM = 64
PDIM = 128
N_HEADS = 26
BATCH = 16384
N_ROWS = BATCH * N_HEADS
NC, NS, L = 2, 16, 16
NW = NC * NS
ROWS_PER_W = N_ROWS // NW
CHUNK = 128
N_CHUNKS = ROWS_PER_W // CHUNK
VREGS_PER_CHUNK = CHUNK // L
NBUF = 4
N_GROUPS = N_CHUNKS // NBUF


def _sc_gather(ids_hbm, offs_hbm, table_hbm, out_hbm,
               idx_v, offs_v, rows_v, gsem, osem):
    wid = lax.axis_index("s") * NC + lax.axis_index("c")
    pltpu.sync_copy(ids_hbm.at[wid], idx_v)
    pltpu.sync_copy(offs_hbm, offs_v)
    out_base = wid * ROWS_PER_W

    def add_offsets(j):
        for k in range(VREGS_PER_CHUNK):
            sl = pl.ds(k * L, L)
            idx_v[j, sl] = idx_v[j, sl] + offs_v[j, sl]

    def gather(j, b):
        return pltpu.make_async_copy(
            table_hbm.at[idx_v.at[j]], rows_v.at[b], gsem.at[b])

    def writeback(j, b):
        return pltpu.make_async_copy(
            rows_v.at[b, :, pl.ds(0, DIM)],
            out_hbm.at[pl.ds(out_base + j * CHUNK, CHUNK)],
            osem.at[b])

    for b in range(NBUF):
        add_offsets(b)
        gather(b, b).start()

    def group_body(g, carry):
        for b in range(NBUF):
            j = g * NBUF + b
            gather(j, b).wait()
            writeback(j, b).start()
            jn = j + NBUF
            add_offsets(jn)
            writeback(j, b).wait()
            gather(jn, b).start()
        return carry

    lax.fori_loop(0, N_GROUPS - 1, group_body, 0)

    for b in range(NBUF):
        j = (N_GROUPS - 1) * NBUF + b
        gather(j, b).wait()
        writeback(j, b).start()
    for b in range(NBUF):
        j = (N_GROUPS - 1) * NBUF + b
        writeback(j, b).wait()


@jax.jit
def _run(ids, offs_tiled, table_padded):
    mesh = plsc.VectorSubcoreMesh(core_axis_name="c", subcore_axis_name="s")
    f = pl.kernel(
        _sc_gather,
        out_type=jax.ShapeDtypeStruct((N_ROWS, DIM), jnp.float32),
        mesh=mesh,
        scratch_types=[
            pltpu.VMEM((N_CHUNKS, CHUNK), jnp.int32),
            pltpu.VMEM((N_CHUNKS, CHUNK), jnp.int32),
            pltpu.VMEM((NBUF, CHUNK, PDIM), jnp.float32),
            pltpu.SemaphoreType.DMA((NBUF,)),
            pltpu.SemaphoreType.DMA((NBUF,)),
        ],
        compiler_params=pltpu.CompilerParams(use_tc_tiling_on_sc=False),
    )
    return f(ids, offs_tiled, table_padded)


def kernel(input_ids, offsets, table):
    ids = input_ids.astype(jnp.int32).reshape(NW, N_CHUNKS, CHUNK)
    offs_tiled = jnp.tile(offsets.astype(jnp.int32),
                          ROWS_PER_W // N_HEADS).reshape(N_CHUNKS, CHUNK)
    tpad = jnp.pad(table.astype(jnp.float32), ((0, 0), (0, PDIM - DIM)))
    out = _run(ids, offs_tiled, tpad)
    return out.reshape(BATCH, N_HEADS, DIM)
